# baseline (speedup 1.0000x reference)
import functools

import jax
import jax.numpy as jnp
from jax import lax
from jax.experimental import pallas as pl
from jax.experimental.pallas import tpu as pltpu
from jax.experimental.pallas import tpu_sc as plsc

N_IDX = 100000
D = 128
C = 128
G = 3
NW = 32
GC = G * C
P_FULL = N_IDX // GC
R_LAST = N_IDX - P_FULL * GC
C_TAIL = R_LAST - C
N_SLOTS = 10
GRID = N_SLOTS * NW * GC

_mesh = plsc.VectorSubcoreMesh(core_axis_name="c", subcore_axis_name="s")


@functools.partial(
    pl.kernel,
    mesh=_mesh,
    out_type=jax.ShapeDtypeStruct((N_IDX, D), jnp.float32),
    scratch_types=[
        pltpu.VMEM((G * N_SLOTS, C), jnp.int32),
        pltpu.VMEM((GC, D), jnp.float32),
        pltpu.VMEM((GC, D), jnp.float32),
        pltpu.SemaphoreType.DMA,
        pltpu.SemaphoreType.DMA,
    ],
)
def _sc_gather(x_hbm, idx3_hbm, out_hbm, idx_v, rows_a, rows_b, gsem, osem):
    w = lax.axis_index("s") * 2 + lax.axis_index("c")
    rows = (rows_a, rows_b)

    pltpu.sync_copy(idx3_hbm.at[w], idx_v)

    def drain_out(n_rows):
        pltpu.make_async_copy(rows_a.at[pl.ds(0, n_rows)],
                              out_hbm.at[pl.ds(0, n_rows)], osem).wait()

    def slot(j, h):
        p = 2 * h + j
        q = p * NW + w
        buf = rows[j % 2]

        @pl.when((q >= 2 * NW) & (q - 2 * NW < P_FULL))
        def _():
            drain_out(GC)

        @pl.when(q - 2 * NW == P_FULL)
        def _():
            drain_out(R_LAST)

        @pl.when(q < P_FULL)
        def _():
            gds = [pltpu.async_copy(x_hbm.at[idx_v.at[G * p + r]],
                                    buf.at[pl.ds(r * C, C)], gsem)
                   for r in range(G)]
            for gd in gds:
                gd.wait()
            pltpu.async_copy(buf, out_hbm.at[pl.ds(q * GC, GC)], osem)

        @pl.when(q == P_FULL)
        def _():
            g0 = pltpu.async_copy(x_hbm.at[idx_v.at[G * p]],
                                  buf.at[pl.ds(0, C)], gsem)
            g1 = pltpu.async_copy(x_hbm.at[idx_v.at[G * p + 1, pl.ds(0, C_TAIL)]],
                                  buf.at[pl.ds(C, C_TAIL)], gsem)
            g0.wait()
            g1.wait()
            pltpu.async_copy(buf.at[pl.ds(0, R_LAST)],
                             out_hbm.at[pl.ds(q * GC, R_LAST)], osem)

    def body(h, carry):
        slot(0, h)
        slot(1, h)
        return carry

    lax.fori_loop(0, N_SLOTS // 2, body, 0)

    for p_last in (N_SLOTS - 2, N_SLOTS - 1):
        q_last = p_last * NW + w

        @pl.when(q_last < P_FULL)
        def _():
            drain_out(GC)

        @pl.when(q_last == P_FULL)
        def _():
            drain_out(R_LAST)


def kernel(x, idx):
    idx32 = idx.astype(jnp.int32)
    idx3 = (jnp.zeros((GRID,), jnp.int32).at[:N_IDX].set(idx32)
            .reshape(N_SLOTS, NW, GC).transpose(1, 0, 2)
            .reshape(NW, G * N_SLOTS, C))
    return _sc_gather(x, idx3)
